# Initial kernel scaffold; baseline (speedup 1.0000x reference)
import jax
import jax.numpy as jnp
from jax.experimental import pallas as pl


def kernel(feats, labels):
    raise NotImplementedError("write your pallas kernel here")


# speedup vs baseline: 1.3721x; 1.3721x over previous
import jax
import jax.numpy as jnp
from jax.experimental import pallas as pl
from jax.experimental.pallas import tpu as pltpu

_POS_A, _POS_B = 1.0, 0.5
_NEG_A, _NEG_B = 0.6, 0.5
_MARGIN, _WEIGHT = 0.1, 0.5
_EPS = 1e-5

_BR = 256


def _dbml_kernel(f_row_ref, f_all_ref, lab_row_ref, lab_col_ref, out_ref):
    f_row = f_row_ref[...]
    f_all = f_all_ref[...]
    sim = jax.lax.dot_general(
        f_row, f_all, (((1,), (1,)), ((), ())),
        preferred_element_type=jnp.float32)

    lab_row = lab_row_ref[...]
    lab_col = lab_col_ref[...]
    same = lab_row == lab_col
    pos_mask = same & (sim < 1.0 - _EPS)
    neg_mask = jnp.logical_not(same)

    B = jnp.float32(sim.shape[1])
    ninf = jnp.float32(-jnp.inf)
    pinf = jnp.float32(jnp.inf)

    mean_all = jnp.sum(sim, axis=1, keepdims=True) / B
    d_all = sim - mean_all
    sigma_all = jnp.sum(d_all * d_all, axis=1, keepdims=True)

    max_neg = jnp.max(jnp.where(neg_mask, sim, ninf), axis=1, keepdims=True)
    min_pos = jnp.min(jnp.where(pos_mask, sim, pinf), axis=1, keepdims=True)

    pos_sel = pos_mask & (sim - _MARGIN < max_neg)
    neg_sel = neg_mask & (sim + _MARGIN > min_pos)
    sel = pos_sel | neg_sel

    one = jnp.float32(1.0)
    zero = jnp.float32(0.0)
    n_pos = jnp.sum(jnp.where(pos_sel, one, zero), axis=1, keepdims=True)
    n_neg = jnp.sum(jnp.where(neg_sel, one, zero), axis=1, keepdims=True)
    valid = (n_pos > 0) & (n_neg > 0)

    cnt = jnp.maximum(n_pos + n_neg, 1.0)
    mean_sel = jnp.sum(jnp.where(sel, sim, zero), axis=1, keepdims=True) / cnt
    d_sel = sim - mean_sel
    sigma_sel = jnp.sum(jnp.where(sel, d_sel * d_sel, zero),
                        axis=1, keepdims=True) / cnt

    t = jnp.exp(2.0 * sim)
    fp = 1.0 + jnp.float32(jnp.e ** 2) * jnp.sum(
        jnp.where(pos_sel, 1.0 / t, zero), axis=1, keepdims=True)
    fn = 1.0 + jnp.exp(jnp.float32(-_NEG_A / _NEG_B)) * jnp.sum(
        jnp.where(neg_sel, t, zero), axis=1, keepdims=True)

    loss = (jnp.log(fp) + jnp.log(fn)
            + _WEIGHT * (jnp.abs(mean_all - mean_sel)
                         + jnp.abs(sigma_all - sigma_sel)))
    out_ref[...] = jnp.where(valid, loss, zero)


def kernel(feats, labels):
    B, D = feats.shape
    labels = labels.astype(jnp.int32)
    lab_row = labels.reshape(B, 1)
    lab_col = labels.reshape(1, B)
    grid = (B // _BR,)
    out = pl.pallas_call(
        _dbml_kernel,
        grid=grid,
        in_specs=[
            pl.BlockSpec((_BR, D), lambda i: (i, 0)),
            pl.BlockSpec((B, D), lambda i: (0, 0)),
            pl.BlockSpec((_BR, 1), lambda i: (i, 0)),
            pl.BlockSpec((1, B), lambda i: (0, 0)),
        ],
        out_specs=pl.BlockSpec((_BR, 1), lambda i: (i, 0)),
        out_shape=jax.ShapeDtypeStruct((B, 1), jnp.float32),
        compiler_params=pltpu.CompilerParams(
            dimension_semantics=("parallel",)),
    )(feats, feats, lab_row, lab_col)
    return jnp.sum(out) / B


# speedup vs baseline: 1.5958x; 1.1631x over previous
import jax
import jax.numpy as jnp
from jax.experimental import pallas as pl
from jax.experimental.pallas import tpu as pltpu

_POS_A, _POS_B = 1.0, 0.5
_NEG_A, _NEG_B = 0.6, 0.5
_MARGIN, _WEIGHT = 0.1, 0.5
_EPS = 1e-5

_BR = 256


def _dbml_kernel(f_row_ref, f_all_ref, lab_row_ref, lab_col_ref, out_ref):
    f_row = f_row_ref[...]
    f_all = f_all_ref[...]
    sim = jax.lax.dot_general(
        f_row, f_all, (((1,), (1,)), ((), ())),
        preferred_element_type=jnp.float32)

    lab_row = lab_row_ref[...]
    lab_col = lab_col_ref[...]
    same = lab_row == lab_col
    pos_mask = same & (sim < 1.0 - _EPS)

    B = jnp.float32(sim.shape[1])
    ninf = jnp.float32(-jnp.inf)
    pinf = jnp.float32(jnp.inf)
    zero = jnp.float32(0.0)
    one = jnp.float32(1.0)

    sim2 = sim * sim
    sum_all = jnp.sum(sim, axis=1, keepdims=True)
    sumsq_all = jnp.sum(sim2, axis=1, keepdims=True)
    mean_all = sum_all / B
    sigma_all = sumsq_all - B * mean_all * mean_all

    max_neg = jnp.max(jnp.where(same, ninf, sim), axis=1, keepdims=True)
    min_pos = jnp.min(jnp.where(pos_mask, sim, pinf), axis=1, keepdims=True)

    pos_sel = pos_mask & (sim < max_neg + _MARGIN)
    neg_sel = (sim > min_pos - _MARGIN) & jnp.logical_not(same)
    sel = pos_sel | neg_sel

    valid = min_pos < max_neg + _MARGIN

    cnt = jnp.maximum(
        jnp.sum(jnp.where(sel, one, zero), axis=1, keepdims=True), 1.0)
    mean_sel = jnp.sum(jnp.where(sel, sim, zero), axis=1, keepdims=True) / cnt
    sumsq_sel = jnp.sum(jnp.where(sel, sim2, zero), axis=1, keepdims=True)
    sigma_sel = sumsq_sel / cnt - mean_sel * mean_sel

    t = jnp.exp(2.0 * sim)
    fp = 1.0 + jnp.float32(jnp.e ** 2) * jnp.sum(
        jnp.where(pos_sel, 1.0 / t, zero), axis=1, keepdims=True)
    fn = 1.0 + jnp.exp(jnp.float32(-_NEG_A / _NEG_B)) * jnp.sum(
        jnp.where(neg_sel, t, zero), axis=1, keepdims=True)

    loss = (jnp.log(fp) + jnp.log(fn)
            + _WEIGHT * (jnp.abs(mean_all - mean_sel)
                         + jnp.abs(sigma_all - sigma_sel)))
    out_ref[...] = jnp.where(valid, loss, zero)


def kernel(feats, labels):
    B, D = feats.shape
    labels = labels.astype(jnp.int32)
    lab_row = labels.reshape(B, 1)
    lab_col = labels.reshape(1, B)
    grid = (B // _BR,)
    out = pl.pallas_call(
        _dbml_kernel,
        grid=grid,
        in_specs=[
            pl.BlockSpec((_BR, D), lambda i: (i, 0)),
            pl.BlockSpec((B, D), lambda i: (0, 0)),
            pl.BlockSpec((_BR, 1), lambda i: (i, 0)),
            pl.BlockSpec((1, B), lambda i: (0, 0)),
        ],
        out_specs=pl.BlockSpec((_BR, 1), lambda i: (i, 0)),
        out_shape=jax.ShapeDtypeStruct((B, 1), jnp.float32),
        compiler_params=pltpu.CompilerParams(
            dimension_semantics=("parallel",)),
    )(feats, feats, lab_row, lab_col)
    return jnp.sum(out) / B


# speedup vs baseline: 1.6947x; 1.0619x over previous
import jax
import jax.numpy as jnp
from jax.experimental import pallas as pl
from jax.experimental.pallas import tpu as pltpu

_POS_A, _POS_B = 1.0, 0.5
_NEG_A, _NEG_B = 0.6, 0.5
_MARGIN, _WEIGHT = 0.1, 0.5
_EPS = 1e-5

_BR = 512


def _dbml_kernel(f_row_ref, f_all_ref, lab_row_ref, lab_col_ref, out_ref):
    f_row = f_row_ref[...]
    f_all = f_all_ref[...]
    sim = jax.lax.dot_general(
        f_row, f_all, (((1,), (1,)), ((), ())),
        preferred_element_type=jnp.float32)

    lab_row = lab_row_ref[...]
    lab_col = lab_col_ref[...]
    same = lab_row == lab_col
    pos_mask = same & (sim < 1.0 - _EPS)

    B = jnp.float32(sim.shape[1])
    ninf = jnp.float32(-jnp.inf)
    pinf = jnp.float32(jnp.inf)
    zero = jnp.float32(0.0)
    one = jnp.float32(1.0)

    sim2 = sim * sim
    sum_all = jnp.sum(sim, axis=1, keepdims=True)
    sumsq_all = jnp.sum(sim2, axis=1, keepdims=True)
    mean_all = sum_all / B
    sigma_all = sumsq_all - B * mean_all * mean_all

    max_neg = jnp.max(jnp.where(same, ninf, sim), axis=1, keepdims=True)
    min_pos = jnp.min(jnp.where(pos_mask, sim, pinf), axis=1, keepdims=True)

    pos_sel = pos_mask & (sim < max_neg + _MARGIN)
    neg_sel = (sim > min_pos - _MARGIN) & jnp.logical_not(same)
    sel = pos_sel | neg_sel

    valid = min_pos < max_neg + _MARGIN

    cnt = jnp.maximum(
        jnp.sum(jnp.where(sel, one, zero), axis=1, keepdims=True), 1.0)
    mean_sel = jnp.sum(jnp.where(sel, sim, zero), axis=1, keepdims=True) / cnt
    sumsq_sel = jnp.sum(jnp.where(sel, sim2, zero), axis=1, keepdims=True)
    sigma_sel = sumsq_sel / cnt - mean_sel * mean_sel

    t = jnp.exp(2.0 * sim)
    fp = 1.0 + jnp.float32(jnp.e ** 2) * jnp.sum(
        jnp.where(pos_sel, 1.0 / t, zero), axis=1, keepdims=True)
    fn = 1.0 + jnp.exp(jnp.float32(-_NEG_A / _NEG_B)) * jnp.sum(
        jnp.where(neg_sel, t, zero), axis=1, keepdims=True)

    loss = (jnp.log(fp) + jnp.log(fn)
            + _WEIGHT * (jnp.abs(mean_all - mean_sel)
                         + jnp.abs(sigma_all - sigma_sel)))
    out_ref[...] = jnp.where(valid, loss, zero)


def kernel(feats, labels):
    B, D = feats.shape
    labels = labels.astype(jnp.int32)
    lab_row = labels.reshape(B, 1)
    lab_col = labels.reshape(1, B)
    grid = (B // _BR,)
    out = pl.pallas_call(
        _dbml_kernel,
        grid=grid,
        in_specs=[
            pl.BlockSpec((_BR, D), lambda i: (i, 0)),
            pl.BlockSpec((B, D), lambda i: (0, 0)),
            pl.BlockSpec((_BR, 1), lambda i: (i, 0)),
            pl.BlockSpec((1, B), lambda i: (0, 0)),
        ],
        out_specs=pl.BlockSpec((_BR, 1), lambda i: (i, 0)),
        out_shape=jax.ShapeDtypeStruct((B, 1), jnp.float32),
        compiler_params=pltpu.CompilerParams(
            dimension_semantics=("parallel",)),
    )(feats, feats, lab_row, lab_col)
    return jnp.sum(out) / B


# speedup vs baseline: 1.7192x; 1.0145x over previous
import jax
import jax.numpy as jnp
from jax.experimental import pallas as pl
from jax.experimental.pallas import tpu as pltpu

_POS_A, _POS_B = 1.0, 0.5
_NEG_A, _NEG_B = 0.6, 0.5
_MARGIN, _WEIGHT = 0.1, 0.5
_EPS = 1e-5

_BR = 512


def _dbml_kernel(f_all_ref, lab_row_ref, lab_col_ref, out_ref, acc_ref):
    i = pl.program_id(0)
    f_row = f_all_ref[pl.ds(i * _BR, _BR), :]
    f_all = f_all_ref[...]
    sim = jax.lax.dot_general(
        f_row, f_all, (((1,), (1,)), ((), ())),
        preferred_element_type=jnp.float32)

    lab_row = lab_row_ref[...]
    lab_col = lab_col_ref[...]
    same = lab_row == lab_col
    pos_mask = same & (sim < 1.0 - _EPS)

    B = jnp.float32(sim.shape[1])
    ninf = jnp.float32(-jnp.inf)
    pinf = jnp.float32(jnp.inf)
    zero = jnp.float32(0.0)
    one = jnp.float32(1.0)

    sim2 = sim * sim
    sum_all = jnp.sum(sim, axis=1, keepdims=True)
    sumsq_all = jnp.sum(sim2, axis=1, keepdims=True)
    mean_all = sum_all / B
    sigma_all = sumsq_all - B * mean_all * mean_all

    max_neg = jnp.max(jnp.where(same, ninf, sim), axis=1, keepdims=True)
    min_pos = jnp.min(jnp.where(pos_mask, sim, pinf), axis=1, keepdims=True)

    pos_sel = pos_mask & (sim < max_neg + _MARGIN)
    neg_sel = (sim > min_pos - _MARGIN) & jnp.logical_not(same)
    sel = pos_sel | neg_sel

    valid = min_pos < max_neg + _MARGIN

    cnt = jnp.maximum(
        jnp.sum(jnp.where(sel, one, zero), axis=1, keepdims=True), 1.0)
    mean_sel = jnp.sum(jnp.where(sel, sim, zero), axis=1, keepdims=True) / cnt
    sumsq_sel = jnp.sum(jnp.where(sel, sim2, zero), axis=1, keepdims=True)
    sigma_sel = sumsq_sel / cnt - mean_sel * mean_sel

    t = jnp.exp(2.0 * sim)
    fp = 1.0 + jnp.float32(jnp.e ** 2) * jnp.sum(
        jnp.where(pos_sel, 1.0 / t, zero), axis=1, keepdims=True)
    fn = 1.0 + jnp.exp(jnp.float32(-_NEG_A / _NEG_B)) * jnp.sum(
        jnp.where(neg_sel, t, zero), axis=1, keepdims=True)

    loss = (jnp.log(fp) + jnp.log(fn)
            + _WEIGHT * (jnp.abs(mean_all - mean_sel)
                         + jnp.abs(sigma_all - sigma_sel)))
    block_sum = jnp.sum(jnp.where(valid, loss, zero))

    @pl.when(i == 0)
    def _init():
        acc_ref[0] = zero

    acc_ref[0] += block_sum

    @pl.when(i == pl.num_programs(0) - 1)
    def _fin():
        out_ref[0, 0] = acc_ref[0] / B


def kernel(feats, labels):
    B, D = feats.shape
    labels = labels.astype(jnp.int32)
    lab_row = labels.reshape(B, 1)
    lab_col = labels.reshape(1, B)
    grid = (B // _BR,)
    out = pl.pallas_call(
        _dbml_kernel,
        grid=grid,
        in_specs=[
            pl.BlockSpec((B, D), lambda i: (0, 0)),
            pl.BlockSpec((_BR, 1), lambda i: (i, 0)),
            pl.BlockSpec((1, B), lambda i: (0, 0)),
        ],
        out_specs=pl.BlockSpec(memory_space=pltpu.SMEM),
        out_shape=jax.ShapeDtypeStruct((1, 1), jnp.float32),
        scratch_shapes=[pltpu.SMEM((1,), jnp.float32)],
        compiler_params=pltpu.CompilerParams(
            dimension_semantics=("arbitrary",)),
    )(feats, lab_row, lab_col)
    return out[0, 0]


# speedup vs baseline: 2.2566x; 1.3126x over previous
import jax
import jax.numpy as jnp
from jax.experimental import pallas as pl
from jax.experimental.pallas import tpu as pltpu

_POS_A, _POS_B = 1.0, 0.5
_NEG_A, _NEG_B = 0.6, 0.5
_MARGIN, _WEIGHT = 0.1, 0.5
_EPS = 1e-5

_BR = 512
_BLOCKS = 2


def _dbml_kernel(f_all_ref, lab_col_ref, out_ref, acc_ref):
    i = pl.program_id(0)

    def _block_loss(blk):
        f_row = f_all_ref[pl.ds(blk * _BR, _BR), :]
        f_all = f_all_ref[...]
        sim = jax.lax.dot_general(
            f_row, f_all, (((1,), (1,)), ((), ())),
            preferred_element_type=jnp.float32)

        lab_col = lab_col_ref[...]
        lab_row = jnp.transpose(
            lab_col_ref[0:1, pl.ds(blk * _BR, _BR)])
        same = lab_row == lab_col

        B = jnp.float32(sim.shape[1])
        ninf = jnp.float32(-jnp.inf)
        pinf = jnp.float32(jnp.inf)
        zero = jnp.float32(0.0)

        sim_pos = jnp.where(same, sim, pinf)
        sim_neg = jnp.where(same, ninf, sim)

        sim2 = sim * sim
        sum_all = jnp.sum(sim, axis=1, keepdims=True)
        sumsq_all = jnp.sum(sim2, axis=1, keepdims=True)
        mean_all = sum_all / B
        sigma_all = sumsq_all - B * mean_all * mean_all

        max_neg = jnp.max(sim_neg, axis=1, keepdims=True)
        mn = jnp.min(sim_pos, axis=1, keepdims=True)
        min_pos = jnp.where(mn < 1.0 - _EPS, mn, pinf)

        thr_p = jnp.minimum(max_neg + _MARGIN, 1.0 - _EPS)
        pos_sel = sim_pos < thr_p
        neg_sel = sim_neg > min_pos - _MARGIN
        sel = pos_sel | neg_sel

        valid = min_pos < max_neg + _MARGIN

        cnt = jnp.maximum(
            jnp.sum(sel, axis=1, keepdims=True).astype(jnp.float32), 1.0)
        mean_sel = jnp.sum(jnp.where(sel, sim, zero),
                           axis=1, keepdims=True) / cnt
        sumsq_sel = jnp.sum(jnp.where(sel, sim2, zero),
                            axis=1, keepdims=True)
        sigma_sel = sumsq_sel / cnt - mean_sel * mean_sel

        t = jnp.exp(2.0 * sim)
        fp = 1.0 + jnp.float32(jnp.e ** 2) * jnp.sum(
            jnp.where(pos_sel, 1.0 / t, zero), axis=1, keepdims=True)
        fn = 1.0 + jnp.exp(jnp.float32(-_NEG_A / _NEG_B)) * jnp.sum(
            jnp.where(neg_sel, t, zero), axis=1, keepdims=True)

        loss = (jnp.log(fp) + jnp.log(fn)
                + _WEIGHT * (jnp.abs(mean_all - mean_sel)
                             + jnp.abs(sigma_all - sigma_sel)))
        return jnp.sum(jnp.where(valid, loss, zero))

    step_sum = jnp.float32(0.0)
    for r in range(_BLOCKS):
        step_sum = step_sum + _block_loss(i * _BLOCKS + r)

    @pl.when(i == 0)
    def _init():
        acc_ref[0] = jnp.float32(0.0)

    acc_ref[0] += step_sum

    @pl.when(i == pl.num_programs(0) - 1)
    def _fin():
        out_ref[0, 0] = acc_ref[0] / jnp.float32(f_all_ref.shape[0])


def kernel(feats, labels):
    B, D = feats.shape
    labels = labels.astype(jnp.int32)
    lab_col = labels.reshape(1, B)
    grid = (B // (_BR * _BLOCKS),)
    out = pl.pallas_call(
        _dbml_kernel,
        grid=grid,
        in_specs=[
            pl.BlockSpec((B, D), lambda i: (0, 0)),
            pl.BlockSpec((1, B), lambda i: (0, 0)),
        ],
        out_specs=pl.BlockSpec(memory_space=pltpu.SMEM),
        out_shape=jax.ShapeDtypeStruct((1, 1), jnp.float32),
        scratch_shapes=[pltpu.SMEM((1,), jnp.float32)],
        compiler_params=pltpu.CompilerParams(
            dimension_semantics=("arbitrary",)),
    )(feats, lab_col)
    return out[0, 0]
